# baseline (speedup 1.0000x reference)
import functools

import jax
import jax.numpy as jnp
from jax import lax
from jax.experimental import pallas as pl
from jax.experimental.pallas import tpu as pltpu
from jax.experimental.pallas import tpu_sc as plsc

_D = 64
_PAD = 0
_NC = 2
_NS = 16
_NW = _NC * _NS
_CHUNK = 200
_NBUF = 4
_LOOK = 2
_GOFF = tuple(range(0, _CHUNK - 15, 16)) + ((_CHUNK - 16,)
                                            if _CHUNK % 16 else ())


def _embed_lookup(idx3, table, n_chunks, bsz, seq):
  mesh = plsc.VectorSubcoreMesh(core_axis_name="c", subcore_axis_name="s")

  @functools.partial(
      pl.kernel,
      out_type=jax.ShapeDtypeStruct((bsz, seq, _D), jnp.float32),
      mesh=mesh,
      compiler_params=pltpu.CompilerParams(
          needs_layout_passes=False, use_tc_tiling_on_sc=False),
      scratch_types=[
          pltpu.VMEM((n_chunks, _CHUNK), jnp.int32),
          [pltpu.VMEM((_CHUNK, _D), jnp.float32) for _ in range(_NBUF)],
          pltpu.VMEM((16,), jnp.int32),
          [pltpu.SemaphoreType.DMA for _ in range(_NBUF)],
          [pltpu.SemaphoreType.DMA for _ in range(_NBUF)],
      ],
  )
  def run(idx_hbm, table_hbm, out_hbm, idx_v, rows, flag_v, gsems, osems):
    wid = lax.axis_index("s") * _NC + lax.axis_index("c")
    pltpu.sync_copy(idx_hbm.at[wid], idx_v)

    def fire(j, s):
      pltpu.make_async_copy(table_hbm.at[idx_v.at[j]], rows[s], gsems[s]).start()

    def out_start(j, s):
      pltpu.make_async_copy(rows[s], out_hbm.at[wid * n_chunks + j],
                            osems[s]).start()

    def out_wait(j, s):
      pltpu.make_async_copy(rows[s], out_hbm.at[wid * n_chunks + j],
                            osems[s]).wait()

    def handle(j, s):
      pltpu.make_async_copy(table_hbm.at[idx_v.at[j]], rows[s], gsems[s]).wait()
      idx_row = idx_v.at[j]
      msk_acc = idx_row[pl.ds(_GOFF[0], 16)] == _PAD
      for off in _GOFF[1:]:
        msk_acc = msk_acc | (idx_row[pl.ds(off, 16)] == _PAD)
      flag_v[...] = jnp.zeros((16,), jnp.int32)
      plsc.store_scatter(flag_v.at[...], [jnp.zeros((16,), jnp.int32)],
                         jnp.ones((16,), jnp.int32), mask=msk_acc)
      nz = flag_v[...][0]

      @pl.when(nz != 0)
      def _fixup():
        zero16 = jnp.zeros((16,), jnp.float32)
        for off in _GOFF:
          v = idx_row[pl.ds(off, 16)]
          msk = v == _PAD
          rowv = off + lax.iota(jnp.int32, 16)

          def cbody(c, carry, _rowv=rowv, _msk=msk):
            colv = jnp.zeros((16,), jnp.int32) + c
            plsc.store_scatter(rows[s].at[...], [_rowv, colv], zero16,
                               mask=_msk)
            return carry

          lax.fori_loop(0, _D, cbody, 0)

      out_start(j, s)

    for j in range(_LOOK):
      fire(j, j % _NBUF)

    def body4(t, carry):
      for b in range(_NBUF):
        j = _NBUF * t + b
        handle(j, b)
        g = j + _LOOK
        s2 = (b + _LOOK) % _NBUF

        @pl.when(g < n_chunks)
        def _next(_g=g, _s2=s2):
          @pl.when(_g >= _NBUF)
          def _drain():
            out_wait(_g - _NBUF, _s2)

          fire(_g, _s2)

      return carry

    lax.fori_loop(0, n_chunks // _NBUF, body4, 0)

    for b in range(_NBUF):
      out_wait(n_chunks - _NBUF + b, b)

  return run(idx3, table)


def kernel(text, text_lengths, embedding_weight):
  del text_lengths
  b, s = text.shape
  assert s == _CHUNK and b % (_NW * _NBUF) == 0
  n_chunks = b // _NW
  idx3 = text.reshape(_NW, n_chunks, _CHUNK).astype(jnp.int32)
  return _embed_lookup(idx3, embedding_weight, n_chunks, b, s)
